# baseline (speedup 1.0000x reference)
import functools

import jax
import jax.numpy as jnp
from jax import lax
from jax.experimental import pallas as pl
from jax.experimental.pallas import tpu as pltpu
from jax.experimental.pallas import tpu_sc as plsc

NC = 2
NS = 16
NW = NC * NS

CHUNK = 125
GROUP = 8
ZROWS = 40


@functools.lru_cache(maxsize=None)
def _make_sc_agg(n, e, f):
    rows_all = e // CHUNK
    rpw = rows_all // NW
    ngroups = rpw // GROUP
    assert ngroups * GROUP == rpw
    rpt = n // NS
    mesh = plsc.VectorSubcoreMesh(
        core_axis_name="c", subcore_axis_name="s", num_cores=NC, num_subcores=NS
    )

    NB = 3

    @functools.partial(
        pl.kernel,
        out_type=jax.ShapeDtypeStruct((NC, n, f), jnp.float32),
        mesh=mesh,
        compiler_params=pltpu.CompilerParams(use_tc_tiling_on_sc=False),
        scratch_types=[
            [pltpu.VMEM((GROUP, CHUNK), jnp.int32) for _ in range(NB)],
            [pltpu.VMEM((GROUP, CHUNK), jnp.int32) for _ in range(NB)],
            [pltpu.VMEM((GROUP, CHUNK, f), jnp.float32) for _ in range(NB)],
            pltpu.VMEM((ZROWS, f), jnp.float32),
            pltpu.VMEM_SHARED((n, f), jnp.float32),
            [pltpu.SemaphoreType.DMA for _ in range(NB)],
            [pltpu.SemaphoreType.DMA for _ in range(NB)],
            [pltpu.SemaphoreType.DMA for _ in range(NB)],
        ],
    )
    def sc_agg(src_hbm, dst_hbm, hw_hbm, out_hbm, srcb, dstb, rowsb,
               zero_v, agg_sh, isem, gsem, ssem):
        c = lax.axis_index("c")
        s = lax.axis_index("s")
        tid = c * NS + s
        rbase = tid * rpw

        idesc, gdesc, sdesc = {}, {}, {}

        def fire_idx(g):
            b = g % NB
            r0 = rbase + g * GROUP
            idesc[g] = (
                pltpu.async_copy(src_hbm.at[pl.ds(r0, GROUP)], srcb[b], isem[b]),
                pltpu.async_copy(dst_hbm.at[pl.ds(r0, GROUP)], dstb[b], isem[b]),
            )

        def fire_gath(g):
            b = g % NB
            for d in idesc[g]:
                d.wait()
            gdesc[g] = [
                pltpu.async_copy(hw_hbm.at[srcb[b].at[j]], rowsb[b].at[j],
                                 gsem[b])
                for j in range(GROUP)
            ]

        def fire_scat(g):
            b = g % NB
            for d in gdesc[g]:
                d.wait()
            sdesc[g] = [
                pltpu.async_copy(rowsb[b].at[j], agg_sh.at[dstb[b].at[j]],
                                 ssem[b], add=True)
                for j in range(GROUP)
            ]

        fire_idx(0)
        fire_gath(0)
        fire_idx(1)

        zero16 = jnp.zeros((16,), jnp.float32)
        for i in range(ZROWS):
            for j in range(f // 16):
                zero_v[i, pl.ds(j * 16, 16)] = zero16
        for k in range(rpt // ZROWS):
            pltpu.sync_copy(zero_v, agg_sh.at[pl.ds(s * rpt + k * ZROWS, ZROWS)])
        plsc.subcore_barrier()

        for g in range(ngroups):
            if g >= 2:
                for d in sdesc.pop(g - 2):
                    d.wait()
            if 2 <= g + 1 < ngroups:
                fire_idx(g + 1)
            if g > 0:
                fire_gath(g)
            fire_scat(g)
        for g in (ngroups - 2, ngroups - 1):
            for d in sdesc.pop(g):
                d.wait()
        plsc.subcore_barrier()

        pltpu.sync_copy(agg_sh.at[pl.ds(s * rpt, rpt)],
                        out_hbm.at[c, pl.ds(s * rpt, rpt)])

    return sc_agg


def _tc_in_matmul(x, w, m_out, bm=512):
    m, k = x.shape
    f = w.shape[1]

    def body(x_ref, w_ref, o_ref):
        o_ref[...] = jnp.dot(x_ref[...], w_ref[...],
                             preferred_element_type=jnp.float32)

    return pl.pallas_call(
        body,
        grid=(pl.cdiv(m_out, bm),),
        in_specs=[
            pl.BlockSpec((bm, k), lambda i: (i, 0)),
            pl.BlockSpec((k, f), lambda i: (0, 0)),
        ],
        out_specs=pl.BlockSpec((bm, f), lambda i: (i, 0)),
        out_shape=jax.ShapeDtypeStruct((m_out, f), jnp.float32),
    )(x, w)


def _tc_fold_matmul(agg, w, bm=512):
    _, m, k = agg.shape
    f = w.shape[1]

    def body(a_ref, w_ref, o_ref):
        h = jnp.maximum(a_ref[0] + a_ref[1], 0.0)
        o_ref[...] = jnp.dot(h, w_ref[...], preferred_element_type=jnp.float32)

    return pl.pallas_call(
        body,
        grid=(pl.cdiv(m, bm),),
        in_specs=[
            pl.BlockSpec((NC, bm, k), lambda i: (0, i, 0)),
            pl.BlockSpec((k, f), lambda i: (0, 0)),
        ],
        out_specs=pl.BlockSpec((bm, f), lambda i: (i, 0)),
        out_shape=jax.ShapeDtypeStruct((m, f), jnp.float32),
    )(agg, w)


def _tc_decoder(aggz, m, bm=512):
    _, _, zd = aggz.shape

    def body(ar_ref, ac_ref, o_ref):
        zi = ar_ref[0] + ar_ref[1]
        zj = ac_ref[0] + ac_ref[1]
        o_ref[...] = lax.dot_general(zi, zj, (((1,), (1,)), ((), ())),
                                     preferred_element_type=jnp.float32)

    out = pl.pallas_call(
        body,
        grid=(pl.cdiv(m, bm), pl.cdiv(m, bm)),
        in_specs=[
            pl.BlockSpec((NC, bm, zd), lambda i, j: (0, i, 0)),
            pl.BlockSpec((NC, bm, zd), lambda i, j: (0, j, 0)),
        ],
        out_specs=pl.BlockSpec((bm, bm), lambda i, j: (i, j)),
        out_shape=jax.ShapeDtypeStruct((m, m), jnp.float32),
    )(aggz, aggz)
    return out.reshape(-1)


def kernel(features, edge_index, W0, W1, W2, W3, W4, W5, W6):
    n = features.shape[0]
    e = edge_index.shape[1]
    n_pad = ((n + 8 * NS - 1) // (8 * NS)) * (8 * NS)
    src = edge_index[0].reshape(e // CHUNK, CHUNK)
    dst = edge_index[1].reshape(e // CHUNK, CHUNK)

    hw = _tc_in_matmul(features, W0, n_pad)
    for w_next in (W1, W2, W3, W4, W5, W6):
        agg = _make_sc_agg(n_pad, e, hw.shape[1])(src, dst, hw)
        hw = _tc_fold_matmul(agg, w_next)
    aggz = _make_sc_agg(n_pad, e, hw.shape[1])(src, dst, hw)
    return _tc_decoder(aggz, n)


# speedup vs baseline: 1.2702x; 1.2702x over previous
import functools

import jax
import jax.numpy as jnp
from jax import lax
from jax.experimental import pallas as pl
from jax.experimental.pallas import tpu as pltpu
from jax.experimental.pallas import tpu_sc as plsc

NC = 2
NS = 16
NW = NC * NS

CHUNK = 125
GROUP = 8
ZROWS = 40


@functools.lru_cache(maxsize=None)
def _make_sc_agg(n, e, f):
    rows_all = e // CHUNK
    rpw = rows_all // NW
    ngroups = rpw // GROUP
    assert ngroups * GROUP == rpw
    rpt = n // NS
    mesh = plsc.VectorSubcoreMesh(
        core_axis_name="c", subcore_axis_name="s", num_cores=NC, num_subcores=NS
    )

    NB = 3

    @functools.partial(
        pl.kernel,
        out_type=jax.ShapeDtypeStruct((NC, n, f), jnp.float32),
        mesh=mesh,
        compiler_params=pltpu.CompilerParams(use_tc_tiling_on_sc=False),
        scratch_types=[
            [pltpu.VMEM((GROUP, CHUNK), jnp.int32) for _ in range(NB)],
            [pltpu.VMEM((GROUP, CHUNK), jnp.int32) for _ in range(NB)],
            [pltpu.VMEM((GROUP, CHUNK, f), jnp.float32) for _ in range(NB)],
            pltpu.VMEM((ZROWS, f), jnp.float32),
            pltpu.VMEM_SHARED((n, f), jnp.float32),
            [pltpu.SemaphoreType.DMA for _ in range(NB)],
            [pltpu.SemaphoreType.DMA for _ in range(NB)],
            [pltpu.SemaphoreType.DMA for _ in range(NB)],
        ],
    )
    def sc_agg(src_hbm, dst_hbm, hw_hbm, out_hbm, srcb, dstb, rowsb,
               zero_v, agg_sh, isem, gsem, ssem):
        c = lax.axis_index("c")
        s = lax.axis_index("s")
        tid = c * NS + s
        rbase = tid * rpw

        idesc, gdesc, sdesc = {}, {}, {}

        def fire_idx(g):
            b = g % NB
            r0 = rbase + g * GROUP
            idesc[g] = (
                pltpu.async_copy(src_hbm.at[pl.ds(r0, GROUP)], srcb[b], isem[b]),
                pltpu.async_copy(dst_hbm.at[pl.ds(r0, GROUP)], dstb[b], isem[b]),
            )

        def fire_gath(g):
            b = g % NB
            for d in idesc[g]:
                d.wait()
            gdesc[g] = [
                pltpu.async_copy(hw_hbm.at[srcb[b].at[j]], rowsb[b].at[j],
                                 gsem[b])
                for j in range(GROUP)
            ]

        def fire_scat(g):
            b = g % NB
            for d in gdesc[g]:
                d.wait()
            sdesc[g] = [
                pltpu.async_copy(rowsb[b].at[j], agg_sh.at[dstb[b].at[j]],
                                 ssem[b], add=True)
                for j in range(GROUP)
            ]

        fire_idx(0)
        fire_gath(0)
        fire_idx(1)

        zero16 = jnp.zeros((16,), jnp.float32)
        for i in range(ZROWS):
            for j in range(f // 16):
                zero_v[i, pl.ds(j * 16, 16)] = zero16
        for k in range(rpt // ZROWS):
            pltpu.sync_copy(zero_v, agg_sh.at[pl.ds(s * rpt + k * ZROWS, ZROWS)])
        plsc.subcore_barrier()

        for g in range(ngroups):
            if g >= 2:
                for d in sdesc.pop(g - 2):
                    d.wait()
            if 2 <= g + 1 < ngroups:
                fire_idx(g + 1)
            if g > 0:
                fire_gath(g)
            fire_scat(g)
        for g in (ngroups - 2, ngroups - 1):
            for d in sdesc.pop(g):
                d.wait()
        plsc.subcore_barrier()

        pltpu.sync_copy(agg_sh.at[pl.ds(s * rpt, rpt)],
                        out_hbm.at[c, pl.ds(s * rpt, rpt)])

    return sc_agg


def _tc_in_matmul(x, w, m_out, bm=512):
    m, k = x.shape
    f = w.shape[1]

    def body(x_ref, w_ref, o_ref):
        o_ref[...] = jnp.dot(x_ref[...], w_ref[...],
                             preferred_element_type=jnp.float32)

    return pl.pallas_call(
        body,
        grid=(pl.cdiv(m_out, bm),),
        in_specs=[
            pl.BlockSpec((bm, k), lambda i: (i, 0)),
            pl.BlockSpec((k, f), lambda i: (0, 0)),
        ],
        out_specs=pl.BlockSpec((bm, f), lambda i: (i, 0)),
        out_shape=jax.ShapeDtypeStruct((m_out, f), jnp.float32),
    )(x, w)


def _tc_fold_matmul(agg, w, bm=512):
    _, m, k = agg.shape
    f = w.shape[1]

    def body(a_ref, w_ref, o_ref):
        h = jnp.maximum(a_ref[0] + a_ref[1], 0.0)
        o_ref[...] = jnp.dot(h, w_ref[...], preferred_element_type=jnp.float32)

    return pl.pallas_call(
        body,
        grid=(pl.cdiv(m, bm),),
        in_specs=[
            pl.BlockSpec((NC, bm, k), lambda i: (0, i, 0)),
            pl.BlockSpec((k, f), lambda i: (0, 0)),
        ],
        out_specs=pl.BlockSpec((bm, f), lambda i: (i, 0)),
        out_shape=jax.ShapeDtypeStruct((m, f), jnp.float32),
    )(agg, w)


def _tc_decoder(aggz, m, bm=256):
    _, _, zd = aggz.shape

    def body(ar_ref, ac_ref, o_ref):
        zi = ar_ref[0] + ar_ref[1]
        zj = ac_ref[0] + ac_ref[1]
        o_ref[...] = lax.dot_general(zi, zj, (((1,), (1,)), ((), ())),
                                     preferred_element_type=jnp.float32)

    out = pl.pallas_call(
        body,
        grid=(pl.cdiv(m, bm),),
        in_specs=[
            pl.BlockSpec((NC, bm, zd), lambda i: (0, i, 0)),
            pl.BlockSpec((NC, m, zd), lambda i: (0, 0, 0)),
        ],
        out_specs=pl.BlockSpec((bm, m), lambda i: (i, 0)),
        out_shape=jax.ShapeDtypeStruct((m, m), jnp.float32),
    )(aggz, aggz)
    return out.reshape(-1)


def kernel(features, edge_index, W0, W1, W2, W3, W4, W5, W6):
    n = features.shape[0]
    e = edge_index.shape[1]
    n_pad = ((n + 8 * NS - 1) // (8 * NS)) * (8 * NS)
    src = edge_index[0].reshape(e // CHUNK, CHUNK)
    dst = edge_index[1].reshape(e // CHUNK, CHUNK)

    hw = _tc_in_matmul(features, W0, n_pad)
    for w_next in (W1, W2, W3, W4, W5, W6):
        agg = _make_sc_agg(n_pad, e, hw.shape[1])(src, dst, hw)
        hw = _tc_fold_matmul(agg, w_next)
    aggz = _make_sc_agg(n_pad, e, hw.shape[1])(src, dst, hw)
    return _tc_decoder(aggz, n)


# speedup vs baseline: 1.3418x; 1.0564x over previous
import functools

import jax
import jax.numpy as jnp
from jax import lax
from jax.experimental import pallas as pl
from jax.experimental.pallas import tpu as pltpu
from jax.experimental.pallas import tpu_sc as plsc

NC = 2
NS = 16
NW = NC * NS

CHUNK = 125
GROUP = 8
ZROWS = 40


@functools.lru_cache(maxsize=None)
def _make_sc_agg(n, e, f):
    rows_all = e // CHUNK
    rpw = rows_all // NW
    ngroups = rpw // GROUP
    assert ngroups * GROUP == rpw
    rpt = n // NS
    mesh = plsc.VectorSubcoreMesh(
        core_axis_name="c", subcore_axis_name="s", num_cores=NC, num_subcores=NS
    )

    NB = 3

    @functools.partial(
        pl.kernel,
        out_type=jax.ShapeDtypeStruct((NC, n, f), jnp.float32),
        mesh=mesh,
        compiler_params=pltpu.CompilerParams(use_tc_tiling_on_sc=False),
        scratch_types=[
            [pltpu.VMEM((GROUP, CHUNK), jnp.int32) for _ in range(NB)],
            [pltpu.VMEM((GROUP, CHUNK), jnp.int32) for _ in range(NB)],
            [pltpu.VMEM((GROUP, CHUNK, f), jnp.float32) for _ in range(NB)],
            pltpu.VMEM((ZROWS, f), jnp.float32),
            pltpu.VMEM_SHARED((n, f), jnp.float32),
            [pltpu.SemaphoreType.DMA for _ in range(NB)],
            [pltpu.SemaphoreType.DMA for _ in range(NB)],
            [pltpu.SemaphoreType.DMA for _ in range(NB)],
        ],
    )
    def sc_agg(src_hbm, dst_hbm, hw_hbm, out_hbm, srcb, dstb, rowsb,
               zero_v, agg_sh, isem, gsem, ssem):
        c = lax.axis_index("c")
        s = lax.axis_index("s")
        tid = c * NS + s
        rbase = tid * rpw

        idesc, gdesc, sdesc = {}, {}, {}

        def fire_idx(g):
            b = g % NB
            r0 = rbase + g * GROUP
            idesc[g] = (
                pltpu.async_copy(src_hbm.at[pl.ds(r0, GROUP)], srcb[b], isem[b]),
                pltpu.async_copy(dst_hbm.at[pl.ds(r0, GROUP)], dstb[b], isem[b]),
            )

        def fire_gath(g):
            b = g % NB
            for d in idesc[g]:
                d.wait()
            gdesc[g] = [
                pltpu.async_copy(hw_hbm.at[srcb[b].at[j]], rowsb[b].at[j],
                                 gsem[b])
                for j in range(GROUP)
            ]

        def fire_scat(g):
            b = g % NB
            for d in gdesc[g]:
                d.wait()
            sdesc[g] = [
                pltpu.async_copy(rowsb[b].at[j], agg_sh.at[dstb[b].at[j]],
                                 ssem[b], add=True)
                for j in range(GROUP)
            ]

        fire_idx(0)
        fire_gath(0)
        fire_idx(1)

        zero16 = jnp.zeros((16,), jnp.float32)
        for i in range(ZROWS):
            for j in range(f // 16):
                zero_v[i, pl.ds(j * 16, 16)] = zero16
        for k in range(rpt // ZROWS):
            pltpu.sync_copy(zero_v, agg_sh.at[pl.ds(s * rpt + k * ZROWS, ZROWS)])
        plsc.subcore_barrier()

        for g in range(ngroups):
            if g >= 2:
                for d in sdesc.pop(g - 2):
                    d.wait()
            if 2 <= g + 1 < ngroups:
                fire_idx(g + 1)
            if g > 0:
                fire_gath(g)
            fire_scat(g)
        for g in (ngroups - 2, ngroups - 1):
            for d in sdesc.pop(g):
                d.wait()
        plsc.subcore_barrier()

        pltpu.sync_copy(agg_sh.at[pl.ds(s * rpt, rpt)],
                        out_hbm.at[c, pl.ds(s * rpt, rpt)])

    return sc_agg


def _tc_in_matmul(x, w, m_out, bm=2048):
    m, k = x.shape
    f = w.shape[1]

    def body(x_ref, w_ref, o_ref):
        o_ref[...] = jnp.dot(x_ref[...], w_ref[...],
                             preferred_element_type=jnp.float32)

    return pl.pallas_call(
        body,
        grid=(pl.cdiv(m_out, bm),),
        in_specs=[
            pl.BlockSpec((bm, k), lambda i: (i, 0)),
            pl.BlockSpec((k, f), lambda i: (0, 0)),
        ],
        out_specs=pl.BlockSpec((bm, f), lambda i: (i, 0)),
        out_shape=jax.ShapeDtypeStruct((m_out, f), jnp.float32),
    )(x, w)


def _tc_fold_matmul(agg, w, bm=2048):
    _, m, k = agg.shape
    f = w.shape[1]

    def body(a_ref, w_ref, o_ref):
        h = jnp.maximum(a_ref[0] + a_ref[1], 0.0)
        o_ref[...] = jnp.dot(h, w_ref[...], preferred_element_type=jnp.float32)

    return pl.pallas_call(
        body,
        grid=(pl.cdiv(m, bm),),
        in_specs=[
            pl.BlockSpec((NC, bm, k), lambda i: (0, i, 0)),
            pl.BlockSpec((k, f), lambda i: (0, 0)),
        ],
        out_specs=pl.BlockSpec((bm, f), lambda i: (i, 0)),
        out_shape=jax.ShapeDtypeStruct((m, f), jnp.float32),
    )(agg, w)


def _tc_decoder(aggz, m, bm=256):
    _, _, zd = aggz.shape

    def body(ar_ref, ac_ref, o_ref):
        zi = ar_ref[0] + ar_ref[1]
        zj = ac_ref[0] + ac_ref[1]
        o_ref[...] = lax.dot_general(zi, zj, (((1,), (1,)), ((), ())),
                                     preferred_element_type=jnp.float32)

    out = pl.pallas_call(
        body,
        grid=(pl.cdiv(m, bm),),
        in_specs=[
            pl.BlockSpec((NC, bm, zd), lambda i: (0, i, 0)),
            pl.BlockSpec((NC, m, zd), lambda i: (0, 0, 0)),
        ],
        out_specs=pl.BlockSpec((bm, m), lambda i: (i, 0)),
        out_shape=jax.ShapeDtypeStruct((m, m), jnp.float32),
    )(aggz, aggz)
    return out.reshape(-1)


def kernel(features, edge_index, W0, W1, W2, W3, W4, W5, W6):
    n = features.shape[0]
    e = edge_index.shape[1]
    n_pad = ((n + 8 * NS - 1) // (8 * NS)) * (8 * NS)
    src = edge_index[0].reshape(e // CHUNK, CHUNK)
    dst = edge_index[1].reshape(e // CHUNK, CHUNK)

    hw = _tc_in_matmul(features, W0, n_pad)
    for w_next in (W1, W2, W3, W4, W5, W6):
        agg = _make_sc_agg(n_pad, e, hw.shape[1])(src, dst, hw)
        hw = _tc_fold_matmul(agg, w_next)
    aggz = _make_sc_agg(n_pad, e, hw.shape[1])(src, dst, hw)
    return _tc_decoder(aggz, n)
